# baseline (speedup 1.0000x reference)
import functools

import jax
import jax.numpy as jnp
from jax import lax
from jax.experimental import pallas as pl
from jax.experimental.pallas import tpu as pltpu
from jax.experimental.pallas import tpu_sc as plsc

WD, TD, PD = 128, 32, 32
OUT_D = WD + TD + PD + PD
NC, NS = 2, 16
NW = NC * NS
NBUF = 2


@functools.partial(jax.jit, static_argnames=("T", "C", "nchunk"))
def _emb_call(idx_w, idx_t, idx_p1, idx_p2, word_table, tag_table, pos_table,
              T, C, nchunk):
    tpw = T // NW
    mesh = plsc.VectorSubcoreMesh(core_axis_name="c", subcore_axis_name="s")

    buf_types = []
    for _ in range(NBUF):
        buf_types += [
            pltpu.VMEM((C, WD), jnp.float32),
            pltpu.VMEM((C, TD), jnp.float32),
            pltpu.VMEM((C, PD), jnp.float32),
            pltpu.VMEM((C, PD), jnp.float32),
            pltpu.SemaphoreType.DMA,
            pltpu.SemaphoreType.DMA,
        ]

    @functools.partial(
        pl.kernel,
        out_type=(
            jax.ShapeDtypeStruct((T, WD), jnp.float32),
            jax.ShapeDtypeStruct((T, TD), jnp.float32),
            jax.ShapeDtypeStruct((T, PD), jnp.float32),
            jax.ShapeDtypeStruct((T, PD), jnp.float32),
        ),
        mesh=mesh,
        scratch_types=[pltpu.VMEM((4, nchunk, C), jnp.int32)] + buf_types,
        compiler_params=pltpu.CompilerParams(use_tc_tiling_on_sc=False),
    )
    def emb(iw_hbm, it_hbm, ip1_hbm, ip2_hbm, wt_hbm, tt_hbm, pt_hbm,
            ow_hbm, ot_hbm, op1_hbm, op2_hbm, idx_v, *bufs):
        slots = [bufs[6 * b:6 * b + 6] for b in range(NBUF)]
        outs = (ow_hbm, ot_hbm, op1_hbm, op2_hbm)
        wid = lax.axis_index("s") * NC + lax.axis_index("c")
        pltpu.sync_copy(iw_hbm.at[wid], idx_v.at[0])
        pltpu.sync_copy(it_hbm.at[wid], idx_v.at[1])
        pltpu.sync_copy(ip1_hbm.at[wid], idx_v.at[2])
        pltpu.sync_copy(ip2_hbm.at[wid], idx_v.at[3])

        def fire(i, b):
            wbuf, tbuf, p1buf, p2buf, gsem, _ = slots[b]
            pltpu.async_copy(wt_hbm.at[idx_v.at[0, i]], wbuf, gsem)
            pltpu.async_copy(tt_hbm.at[idx_v.at[1, i]], tbuf, gsem)
            pltpu.async_copy(pt_hbm.at[idx_v.at[2, i]], p1buf, gsem)
            pltpu.async_copy(pt_hbm.at[idx_v.at[3, i]], p2buf, gsem)

        def drain_gather_fire_write(i, b):
            wbuf, tbuf, p1buf, p2buf, gsem, wsem = slots[b]
            pltpu.make_async_copy(wt_hbm.at[idx_v.at[0, i]], wbuf, gsem).wait()
            pltpu.make_async_copy(tt_hbm.at[idx_v.at[1, i]], tbuf, gsem).wait()
            pltpu.make_async_copy(pt_hbm.at[idx_v.at[2, i]], p1buf, gsem).wait()
            pltpu.make_async_copy(pt_hbm.at[idx_v.at[3, i]], p2buf, gsem).wait()
            base = wid * tpw + i * C
            for buf, o in zip((wbuf, tbuf, p1buf, p2buf), outs):
                pltpu.async_copy(buf, o.at[pl.ds(base, C)], wsem)

        def drain_write(i, b):
            wbuf, tbuf, p1buf, p2buf, _, wsem = slots[b]
            base = wid * tpw + i * C
            for buf, o in zip((wbuf, tbuf, p1buf, p2buf), outs):
                pltpu.make_async_copy(buf, o.at[pl.ds(base, C)], wsem).wait()

        for b in range(NBUF):
            fire(b, b)

        @pl.loop(0, nchunk // NBUF)
        def body(j):
            for b in range(NBUF):
                i = j * NBUF + b
                drain_gather_fire_write(i, b)

                @pl.when(i + NBUF < nchunk)
                def _():
                    drain_write(i, b)
                    fire(i + NBUF, b)

                @pl.when(i + NBUF >= nchunk)
                def _():
                    drain_write(i, b)

    return emb(idx_w, idx_t, idx_p1, idx_p2, word_table, tag_table, pos_table)


def kernel(word_id, tag_id, pos_1, pos_2, word_table, tag_table, pos_table):
    B, L = word_id.shape
    T = B * L
    C = 128
    nchunk = T // (NW * C)
    shape = (NW, nchunk, C)
    w, t, p1, p2 = _emb_call(
        word_id.reshape(shape).astype(jnp.int32),
        tag_id.reshape(shape).astype(jnp.int32),
        pos_1.reshape(shape).astype(jnp.int32),
        pos_2.reshape(shape).astype(jnp.int32),
        word_table, tag_table, pos_table,
        T=T, C=C, nchunk=nchunk,
    )
    return jnp.concatenate([
        w.reshape(B, L, WD), t.reshape(B, L, TD),
        p1.reshape(B, L, PD), p2.reshape(B, L, PD)], axis=-1)


# speedup vs baseline: 1.3528x; 1.3528x over previous
import functools

import jax
import jax.numpy as jnp
from jax import lax
from jax.experimental import pallas as pl
from jax.experimental.pallas import tpu as pltpu
from jax.experimental.pallas import tpu_sc as plsc

WD, TD, PD = 128, 32, 32
OUT_D = WD + TD + PD + PD
NC, NS = 2, 16
NW = NC * NS
NBUF = 2


@functools.partial(jax.jit, static_argnames=("T", "C", "nchunk"))
def _emb_call(idx_w, idx_t, idx_p1, idx_p2, word_table, tag_table, pos_table,
              T, C, nchunk):
    tpw = T // NW
    mesh = plsc.VectorSubcoreMesh(core_axis_name="c", subcore_axis_name="s")

    buf_types = []
    for _ in range(NBUF):
        buf_types += [
            pltpu.VMEM((C, WD), jnp.float32),
            pltpu.VMEM((C, TD), jnp.float32),
            pltpu.VMEM((C, PD), jnp.float32),
            pltpu.VMEM((C, PD), jnp.float32),
            pltpu.SemaphoreType.DMA,
            pltpu.SemaphoreType.DMA,
        ]

    @functools.partial(
        pl.kernel,
        out_type=jax.ShapeDtypeStruct((T, OUT_D), jnp.float32),
        mesh=mesh,
        scratch_types=[pltpu.VMEM((4, nchunk, C), jnp.int32)] + buf_types,
        compiler_params=pltpu.CompilerParams(use_tc_tiling_on_sc=False),
    )
    def emb(iw_hbm, it_hbm, ip1_hbm, ip2_hbm, wt_hbm, tt_hbm, pt_hbm, out_hbm,
            idx_v, *bufs):
        slots = [bufs[6 * b:6 * b + 6] for b in range(NBUF)]
        wid = lax.axis_index("s") * NC + lax.axis_index("c")
        pltpu.sync_copy(iw_hbm.at[wid], idx_v.at[0])
        pltpu.sync_copy(it_hbm.at[wid], idx_v.at[1])
        pltpu.sync_copy(ip1_hbm.at[wid], idx_v.at[2])
        pltpu.sync_copy(ip2_hbm.at[wid], idx_v.at[3])

        def fire(i, b):
            wbuf, tbuf, p1buf, p2buf, gsem, _ = slots[b]
            pltpu.async_copy(wt_hbm.at[idx_v.at[0, i]], wbuf, gsem)
            pltpu.async_copy(tt_hbm.at[idx_v.at[1, i]], tbuf, gsem)
            pltpu.async_copy(pt_hbm.at[idx_v.at[2, i]], p1buf, gsem)
            pltpu.async_copy(pt_hbm.at[idx_v.at[3, i]], p2buf, gsem)

        def drain_gather_fire_write(i, b):
            wbuf, tbuf, p1buf, p2buf, gsem, wsem = slots[b]
            pltpu.make_async_copy(wt_hbm.at[idx_v.at[0, i]], wbuf, gsem).wait()
            pltpu.make_async_copy(tt_hbm.at[idx_v.at[1, i]], tbuf, gsem).wait()
            pltpu.make_async_copy(pt_hbm.at[idx_v.at[2, i]], p1buf, gsem).wait()
            pltpu.make_async_copy(pt_hbm.at[idx_v.at[3, i]], p2buf, gsem).wait()
            base = wid * tpw + i * C
            pltpu.async_copy(wbuf, out_hbm.at[pl.ds(base, C), pl.ds(0, WD)], wsem)
            pltpu.async_copy(tbuf, out_hbm.at[pl.ds(base, C), pl.ds(WD, TD)], wsem)
            pltpu.async_copy(p1buf, out_hbm.at[pl.ds(base, C), pl.ds(WD + TD, PD)], wsem)
            pltpu.async_copy(p2buf, out_hbm.at[pl.ds(base, C), pl.ds(WD + TD + PD, PD)], wsem)

        def drain_write(i, b):
            wbuf, tbuf, p1buf, p2buf, _, wsem = slots[b]
            base = wid * tpw + i * C
            pltpu.make_async_copy(wbuf, out_hbm.at[pl.ds(base, C), pl.ds(0, WD)], wsem).wait()
            pltpu.make_async_copy(tbuf, out_hbm.at[pl.ds(base, C), pl.ds(WD, TD)], wsem).wait()
            pltpu.make_async_copy(p1buf, out_hbm.at[pl.ds(base, C), pl.ds(WD + TD, PD)], wsem).wait()
            pltpu.make_async_copy(p2buf, out_hbm.at[pl.ds(base, C), pl.ds(WD + TD + PD, PD)], wsem).wait()

        for b in range(NBUF):
            fire(b, b)

        @pl.loop(0, nchunk // NBUF)
        def body(j):
            for b in range(NBUF):
                i = j * NBUF + b
                drain_gather_fire_write(i, b)

                @pl.when(i + NBUF < nchunk)
                def _():
                    drain_write(i, b)
                    fire(i + NBUF, b)

                @pl.when(i + NBUF >= nchunk)
                def _():
                    drain_write(i, b)

    return emb(idx_w, idx_t, idx_p1, idx_p2, word_table, tag_table, pos_table)


def kernel(word_id, tag_id, pos_1, pos_2, word_table, tag_table, pos_table):
    B, L = word_id.shape
    T = B * L
    C = 128
    nchunk = T // (NW * C)
    shape = (NW, nchunk, C)
    out = _emb_call(
        word_id.reshape(shape).astype(jnp.int32),
        tag_id.reshape(shape).astype(jnp.int32),
        pos_1.reshape(shape).astype(jnp.int32),
        pos_2.reshape(shape).astype(jnp.int32),
        word_table, tag_table, pos_table,
        T=T, C=C, nchunk=nchunk,
    )
    return out.reshape(B, L, OUT_D)


# speedup vs baseline: 1.3544x; 1.0012x over previous
import functools

import jax
import jax.numpy as jnp
from jax import lax
from jax.experimental import pallas as pl
from jax.experimental.pallas import tpu as pltpu
from jax.experimental.pallas import tpu_sc as plsc

WD, TD, PD = 128, 32, 32
OUT_D = WD + TD + PD + PD
NC, NS = 2, 16
NW = NC * NS
NBUF = 2


@functools.partial(jax.jit, static_argnames=("T", "C", "nchunk"))
def _emb_call(idx_w, idx_t, idx_p1, idx_p2, word_table, tag_table, pos_table,
              T, C, nchunk):
    tpw = T // NW
    mesh = plsc.VectorSubcoreMesh(core_axis_name="c", subcore_axis_name="s")

    buf_types = []
    for _ in range(NBUF):
        buf_types += [
            pltpu.VMEM((C, WD), jnp.float32),
            pltpu.VMEM((C, TD), jnp.float32),
            pltpu.VMEM((C, PD), jnp.float32),
            pltpu.VMEM((C, PD), jnp.float32),
            pltpu.SemaphoreType.DMA,
            pltpu.SemaphoreType.DMA,
        ]

    @functools.partial(
        pl.kernel,
        out_type=jax.ShapeDtypeStruct((T, OUT_D), jnp.float32),
        mesh=mesh,
        scratch_types=[pltpu.VMEM((4, nchunk, C), jnp.int32)] + buf_types,
        compiler_params=pltpu.CompilerParams(use_tc_tiling_on_sc=False, disable_bounds_checks=True, disable_semaphore_checks=True),
    )
    def emb(iw_hbm, it_hbm, ip1_hbm, ip2_hbm, wt_hbm, tt_hbm, pt_hbm, out_hbm,
            idx_v, *bufs):
        slots = [bufs[6 * b:6 * b + 6] for b in range(NBUF)]
        wid = lax.axis_index("s") * NC + lax.axis_index("c")
        pltpu.sync_copy(iw_hbm.at[wid], idx_v.at[0])
        pltpu.sync_copy(it_hbm.at[wid], idx_v.at[1])
        pltpu.sync_copy(ip1_hbm.at[wid], idx_v.at[2])
        pltpu.sync_copy(ip2_hbm.at[wid], idx_v.at[3])

        def fire(i, b):
            wbuf, tbuf, p1buf, p2buf, gsem, _ = slots[b]
            pltpu.async_copy(wt_hbm.at[idx_v.at[0, i]], wbuf, gsem)
            pltpu.async_copy(tt_hbm.at[idx_v.at[1, i]], tbuf, gsem)
            pltpu.async_copy(pt_hbm.at[idx_v.at[2, i]], p1buf, gsem)
            pltpu.async_copy(pt_hbm.at[idx_v.at[3, i]], p2buf, gsem)

        def drain_gather_fire_write(i, b):
            wbuf, tbuf, p1buf, p2buf, gsem, wsem = slots[b]
            pltpu.make_async_copy(wt_hbm.at[idx_v.at[0, i]], wbuf, gsem).wait()
            pltpu.make_async_copy(tt_hbm.at[idx_v.at[1, i]], tbuf, gsem).wait()
            pltpu.make_async_copy(pt_hbm.at[idx_v.at[2, i]], p1buf, gsem).wait()
            pltpu.make_async_copy(pt_hbm.at[idx_v.at[3, i]], p2buf, gsem).wait()
            base = wid * tpw + i * C
            pltpu.async_copy(wbuf, out_hbm.at[pl.ds(base, C), pl.ds(0, WD)], wsem)
            pltpu.async_copy(tbuf, out_hbm.at[pl.ds(base, C), pl.ds(WD, TD)], wsem)
            pltpu.async_copy(p1buf, out_hbm.at[pl.ds(base, C), pl.ds(WD + TD, PD)], wsem)
            pltpu.async_copy(p2buf, out_hbm.at[pl.ds(base, C), pl.ds(WD + TD + PD, PD)], wsem)

        def drain_write(i, b):
            wbuf, tbuf, p1buf, p2buf, _, wsem = slots[b]
            base = wid * tpw + i * C
            pltpu.make_async_copy(wbuf, out_hbm.at[pl.ds(base, C), pl.ds(0, WD)], wsem).wait()
            pltpu.make_async_copy(tbuf, out_hbm.at[pl.ds(base, C), pl.ds(WD, TD)], wsem).wait()
            pltpu.make_async_copy(p1buf, out_hbm.at[pl.ds(base, C), pl.ds(WD + TD, PD)], wsem).wait()
            pltpu.make_async_copy(p2buf, out_hbm.at[pl.ds(base, C), pl.ds(WD + TD + PD, PD)], wsem).wait()

        for b in range(NBUF):
            fire(b, b)

        @pl.loop(0, nchunk // NBUF)
        def body(j):
            for b in range(NBUF):
                i = j * NBUF + b
                drain_gather_fire_write(i, b)

                @pl.when(i + NBUF < nchunk)
                def _():
                    drain_write(i, b)
                    fire(i + NBUF, b)

                @pl.when(i + NBUF >= nchunk)
                def _():
                    drain_write(i, b)

    return emb(idx_w, idx_t, idx_p1, idx_p2, word_table, tag_table, pos_table)


def kernel(word_id, tag_id, pos_1, pos_2, word_table, tag_table, pos_table):
    B, L = word_id.shape
    T = B * L
    C = 128
    nchunk = T // (NW * C)
    shape = (NW, nchunk, C)
    out = _emb_call(
        word_id.reshape(shape).astype(jnp.int32),
        tag_id.reshape(shape).astype(jnp.int32),
        pos_1.reshape(shape).astype(jnp.int32),
        pos_2.reshape(shape).astype(jnp.int32),
        word_table, tag_table, pos_table,
        T=T, C=C, nchunk=nchunk,
    )
    return out.reshape(B, L, OUT_D)
